# baseline (speedup 1.0000x reference)
import functools

import jax
import jax.numpy as jnp
from jax import lax
from jax.experimental import pallas as pl
from jax.experimental.pallas import tpu as pltpu
from jax.experimental.pallas import tpu_sc as plsc

BATCH = 4
SEQ = 8192
DM = 1024
CHUNK = 24
NBUF = 2


@functools.cache
def _sc_copy_kernel():
    info = plsc.get_sparse_core_info()
    nw = info.num_cores * info.num_subcores
    ns = info.num_subcores
    rows_per = SEQ // nw
    sizes = [CHUNK] * (rows_per // CHUNK)
    if rows_per % CHUNK:
        sizes.append(rows_per % CHUNK)
    offs = [sum(sizes[:i]) for i in range(len(sizes))]
    nch = len(sizes)
    mesh = plsc.VectorSubcoreMesh(core_axis_name="c", subcore_axis_name="s")

    @functools.partial(
        pl.kernel,
        mesh=mesh,
        out_type=jax.ShapeDtypeStruct((BATCH, SEQ, DM), jnp.float32),
        scratch_types=(
            [pltpu.VMEM((CHUNK, DM), jnp.float32)] * NBUF
            + [pltpu.VMEM_SHARED((ns, NBUF, CHUNK, DM), jnp.float32)]
            + [pltpu.SemaphoreType.DMA] * (4 * NBUF)
        ),
    )
    def sc_copy(table_hbm, out_hbm, *scratch):
        bufs = scratch[:NBUF]
        spm = scratch[NBUF]
        sems = scratch[NBUF + 1 :]
        lsem = sems[:NBUF]
        ssem = sems[NBUF : 2 * NBUF]
        lsem2 = sems[2 * NBUF : 3 * NBUF]
        ssem2 = sems[3 * NBUF :]
        cid = lax.axis_index("c")
        sid = lax.axis_index("s")
        wid = sid * info.num_cores + cid
        base = wid * rows_per

        def row(k):
            return base + offs[k]

        def buf(k):
            b = bufs[k % NBUF]
            return b if sizes[k] == CHUNK else b.at[pl.ds(0, sizes[k])]

        def sbuf(k):
            return spm.at[sid, k % NBUF, pl.ds(0, sizes[k])]

        loads = {}
        loads2 = {}
        stores = {}
        stores2 = {}
        for j in range(min(NBUF - 1, nch)):
            loads[j] = pltpu.async_copy(
                table_hbm.at[pl.ds(row(j), sizes[j])], buf(j), lsem[j]
            )
            loads2[j] = pltpu.async_copy(
                table_hbm.at[pl.ds(row(j), sizes[j])], sbuf(j), lsem2[j]
            )
        for k in range(nch):
            loads[k].wait()
            stores[k] = [
                pltpu.async_copy(
                    buf(k), out_hbm.at[b, pl.ds(row(k), sizes[k])], ssem[k % NBUF]
                )
                for b in range(2)
            ]
            loads2[k].wait()
            stores2[k] = [
                pltpu.async_copy(
                    sbuf(k), out_hbm.at[b, pl.ds(row(k), sizes[k])], ssem2[k % NBUF]
                )
                for b in range(2, BATCH)
            ]
            nxt = k + NBUF - 1
            if nxt < nch:
                if k >= 1:
                    for h in stores[k - 1]:
                        h.wait()
                    for h in stores2[k - 1]:
                        h.wait()
                loads[nxt] = pltpu.async_copy(
                    table_hbm.at[pl.ds(row(nxt), sizes[nxt])], buf(nxt), lsem[nxt % NBUF]
                )
                loads2[nxt] = pltpu.async_copy(
                    table_hbm.at[pl.ds(row(nxt), sizes[nxt])], sbuf(nxt), lsem2[nxt % NBUF]
                )
        for k in range(max(0, nch - NBUF), nch):
            for h in stores[k]:
                h.wait()
            for h in stores2[k]:
                h.wait()

    return sc_copy


def kernel(x, table):
    del x
    return _sc_copy_kernel()(table)


# speedup vs baseline: 1.2149x; 1.2149x over previous
import functools

import jax
import jax.numpy as jnp
from jax import lax
from jax.experimental import pallas as pl
from jax.experimental.pallas import tpu as pltpu
from jax.experimental.pallas import tpu_sc as plsc

BATCH = 4
SEQ = 8192
DM = 1024
CHUNK = 48


NBUF = 2


@functools.cache
def _sc_copy_kernel():
    info = plsc.get_sparse_core_info()
    nw = info.num_cores * info.num_subcores
    rows_per = SEQ // nw
    sizes = [CHUNK] * (rows_per // CHUNK)
    if rows_per % CHUNK:
        sizes.insert(0, rows_per % CHUNK)
    offs = [sum(sizes[:i]) for i in range(len(sizes))]
    nch = len(sizes)
    mesh = plsc.VectorSubcoreMesh(core_axis_name="c", subcore_axis_name="s")

    @functools.partial(
        pl.kernel,
        mesh=mesh,
        out_type=jax.ShapeDtypeStruct((BATCH, SEQ, DM), jnp.float32),
        scratch_types=(
            [pltpu.VMEM((CHUNK, DM), jnp.float32)] * NBUF
            + [pltpu.SemaphoreType.DMA] * (2 * NBUF)
        ),
    )
    def sc_copy(table_hbm, out_hbm, *scratch):
        bufs = scratch[:NBUF]
        lsem = scratch[NBUF : 2 * NBUF]
        ssem = scratch[2 * NBUF :]
        wid = lax.axis_index("s") * info.num_cores + lax.axis_index("c")
        base = wid * rows_per

        def row(k):
            return base + offs[k]

        def buf(k):
            b = bufs[k % NBUF]
            return b if sizes[k] == CHUNK else b.at[pl.ds(0, sizes[k])]

        loads = {}
        stores = {}
        for j in range(min(NBUF - 1, nch)):
            loads[j] = pltpu.async_copy(
                table_hbm.at[pl.ds(row(j), sizes[j])], buf(j), lsem[j]
            )
        for k in range(nch):
            loads[k].wait()
            stores[k] = [
                pltpu.async_copy(
                    buf(k), out_hbm.at[b, pl.ds(row(k), sizes[k])], ssem[k % NBUF]
                )
                for b in range(BATCH)
            ]
            nxt = k + NBUF - 1
            if nxt < nch:
                if k >= 1:
                    for h in stores[k - 1]:
                        h.wait()
                loads[nxt] = pltpu.async_copy(
                    table_hbm.at[pl.ds(row(nxt), sizes[nxt])], buf(nxt), lsem[nxt % NBUF]
                )
        for k in range(max(0, nch - NBUF), nch):
            for h in stores[k]:
                h.wait()

    return sc_copy


def kernel(x, table):
    del x
    return _sc_copy_kernel()(table)
